# baseline (speedup 1.0000x reference)
import functools

import jax
import jax.numpy as jnp
from jax import lax
from jax.experimental import pallas as pl
from jax.experimental.pallas import tpu as pltpu
from jax.experimental.pallas import tpu_sc as plsc

N = 10000
D = 128
E = 320000
NC = 2
NS = 16
NW = NC * NS
EPW = E // NW
CH = 120
NRING = 82
REM = EPW - (NRING + 1) * CH
RPT = 632
RPT_LAST = N - RPT * (NS - 1)


def _seg_sum_body(h_hbm, src_hbm, dst_hbm, zeros_hbm, out_hbm,
                  sidx_all, didx_all, rows0, rows1, agg_sh, sem0, sem1):
    cid = lax.axis_index("c")
    sid = lax.axis_index("s")
    wid = sid * NC + cid
    rows = (rows0, rows1)
    sems = (sem0, sem1)

    pltpu.sync_copy(src_hbm.at[pl.ds(wid * EPW, EPW)], sidx_all)
    pltpu.sync_copy(dst_hbm.at[pl.ds(wid * EPW, EPW)], didx_all)

    def gather_start(c, b):
        pltpu.async_copy(h_hbm.at[sidx_all.at[pl.ds(c * CH, CH)]],
                         rows[b], sems[b])

    def gather_wait(c, b):
        pltpu.make_async_copy(h_hbm.at[sidx_all.at[pl.ds(c * CH, CH)]],
                              rows[b], sems[b]).wait()

    gather_start(0, 0)
    gather_start(1, 1)

    @pl.when(sid < NS - 1)
    def _():
        pltpu.sync_copy(zeros_hbm.at[pl.ds(sid * RPT, RPT)],
                        agg_sh.at[pl.ds(sid * RPT, RPT)])

    @pl.when(sid == NS - 1)
    def _():
        pltpu.sync_copy(zeros_hbm.at[pl.ds(RPT * (NS - 1), RPT_LAST)],
                        agg_sh.at[pl.ds(RPT * (NS - 1), RPT_LAST)])

    plsc.subcore_barrier()

    def ring(g, _):
        for b in range(2):
            c = g + b
            gather_wait(c, b)

            @pl.when(c + 2 < NRING)
            def _():
                gather_start(c + 2, b)

            pltpu.sync_copy(rows[b], agg_sh.at[didx_all.at[pl.ds(c * CH, CH)]],
                            add=True)
        return ()

    lax.fori_loop(0, NRING // 2, lambda i, c: ring(2 * i, c), (),
                  unroll=False)

    eb = NRING * CH
    rb = eb + CH
    pltpu.async_copy(h_hbm.at[sidx_all.at[pl.ds(eb, CH)]], rows0, sem0)
    pltpu.async_copy(h_hbm.at[sidx_all.at[pl.ds(rb, REM)]],
                     rows1.at[pl.ds(0, REM)], sem1)
    pltpu.make_async_copy(h_hbm.at[sidx_all.at[pl.ds(eb, CH)]], rows0,
                          sem0).wait()
    pltpu.sync_copy(rows0, agg_sh.at[didx_all.at[pl.ds(eb, CH)]], add=True)
    pltpu.make_async_copy(h_hbm.at[sidx_all.at[pl.ds(rb, REM)]],
                          rows1.at[pl.ds(0, REM)], sem1).wait()
    pltpu.sync_copy(rows1.at[pl.ds(0, REM)],
                    agg_sh.at[didx_all.at[pl.ds(rb, REM)]], add=True)

    plsc.subcore_barrier()

    @pl.when(sid < NS - 1)
    def _():
        pltpu.sync_copy(agg_sh.at[pl.ds(sid * RPT, RPT)],
                        out_hbm.at[cid, pl.ds(sid * RPT, RPT)])

    @pl.when(sid == NS - 1)
    def _():
        pltpu.sync_copy(agg_sh.at[pl.ds(RPT * (NS - 1), RPT_LAST)],
                        out_hbm.at[cid, pl.ds(RPT * (NS - 1), RPT_LAST)])


@jax.jit
def _segment_sum_sc(h, src, dst, zeros):
    mesh = plsc.VectorSubcoreMesh(core_axis_name="c", subcore_axis_name="s",
                                  num_cores=NC, num_subcores=NS)
    f = pl.kernel(
        _seg_sum_body,
        out_type=jax.ShapeDtypeStruct((NC, N, D), jnp.float32),
        mesh=mesh,
        scratch_types=[
            pltpu.VMEM((EPW,), jnp.int32),
            pltpu.VMEM((EPW,), jnp.int32),
            pltpu.VMEM((CH, D), jnp.float32),
            pltpu.VMEM((CH, D), jnp.float32),
            pltpu.VMEM_SHARED((N, D), jnp.float32),
            pltpu.SemaphoreType.DMA,
            pltpu.SemaphoreType.DMA,
        ],
    )
    return f(h, src, dst, zeros)


BN = 1000


def _mlp_body(eps_ref, h_ref, agg_ref, Wa_ref, ba_ref, Wb_ref, bb_ref, out_ref):
    ev = eps_ref[0, 0]
    z = h_ref[...] * ev + agg_ref[0] + agg_ref[1]
    z = jnp.maximum(
        jnp.dot(z, Wa_ref[...], preferred_element_type=jnp.float32)
        + ba_ref[...], 0.0)
    z = jnp.maximum(
        jnp.dot(z, Wb_ref[...], preferred_element_type=jnp.float32)
        + bb_ref[...], 0.0)
    out_ref[...] = z


@jax.jit
def _mlp_tc(epsv, h, agg, Wa, ba, Wb, bb):
    grid = (N // BN,)
    return pl.pallas_call(
        _mlp_body,
        grid=grid,
        in_specs=[
            pl.BlockSpec(memory_space=pltpu.SMEM),
            pl.BlockSpec((BN, D), lambda i: (i, 0)),
            pl.BlockSpec((NC, BN, D), lambda i: (0, i, 0)),
            pl.BlockSpec((D, D), lambda i: (0, 0)),
            pl.BlockSpec((1, D), lambda i: (0, 0)),
            pl.BlockSpec((D, D), lambda i: (0, 0)),
            pl.BlockSpec((1, D), lambda i: (0, 0)),
        ],
        out_specs=pl.BlockSpec((BN, D), lambda i: (i, 0)),
        out_shape=jax.ShapeDtypeStruct((N, D), jnp.float32),
    )(epsv, h, agg, Wa, ba, Wb, bb)


F = N * D
BF = 6400
NBF = F // BF
NA = 12
NVH = 64


def _head_body(h_ref, Wadv_ref, Wv1_ref, badv_ref, bv1_ref, Wv2_ref, bv2_ref,
               Wv3t_ref, bv3_ref, out_ref, acc_adv, acc_val):
    i = pl.program_id(0)

    @pl.when(i == 0)
    def _():
        acc_adv[...] = jnp.zeros_like(acc_adv)
        acc_val[...] = jnp.zeros_like(acc_val)

    hb = h_ref[...]
    acc_adv[...] += jnp.dot(hb, Wadv_ref[...], preferred_element_type=jnp.float32)
    acc_val[...] += jnp.dot(hb, Wv1_ref[...], preferred_element_type=jnp.float32)

    @pl.when(i == NBF - 1)
    def _():
        adv = jnp.maximum(acc_adv[...] + badv_ref[...], 0.0)
        val = jnp.maximum(acc_val[...] + bv1_ref[...], 0.0)
        val = jnp.maximum(
            jnp.dot(val, Wv2_ref[...], preferred_element_type=jnp.float32)
            + bv2_ref[...], 0.0)
        v3 = jnp.sum(val * Wv3t_ref[...], axis=1, keepdims=True) + bv3_ref[...]
        r = lax.broadcasted_iota(jnp.int32, (NA, NA), 0)
        c = lax.broadcasted_iota(jnp.int32, (NA, NA), 1)
        G = jnp.where((r // 4) == (c // 4), 0.25, 0.0).astype(jnp.float32)
        madv = jnp.dot(adv, G, preferred_element_type=jnp.float32)
        out_ref[...] = v3 + adv - madv


@jax.jit
def _head_tc(h_flat, Wadv, Wv1, badv, bv1, Wv2, bv2, Wv3t, bv3):
    return pl.pallas_call(
        _head_body,
        grid=(NBF,),
        in_specs=[
            pl.BlockSpec((1, BF), lambda i: (0, i)),
            pl.BlockSpec((BF, NA), lambda i: (i, 0)),
            pl.BlockSpec((BF, NVH), lambda i: (i, 0)),
            pl.BlockSpec((1, NA), lambda i: (0, 0)),
            pl.BlockSpec((1, NVH), lambda i: (0, 0)),
            pl.BlockSpec((NVH, NVH), lambda i: (0, 0)),
            pl.BlockSpec((1, NVH), lambda i: (0, 0)),
            pl.BlockSpec((1, NVH), lambda i: (0, 0)),
            pl.BlockSpec((1, 1), lambda i: (0, 0)),
        ],
        out_specs=pl.BlockSpec((1, NA), lambda i: (0, 0)),
        out_shape=jax.ShapeDtypeStruct((1, NA), jnp.float32),
        scratch_shapes=[
            pltpu.VMEM((1, NA), jnp.float32),
            pltpu.VMEM((1, NVH), jnp.float32),
        ],
    )(h_flat, Wadv, Wv1, badv, bv1, Wv2, bv2, Wv3t, bv3)


def kernel(x, edge_index, W1a, b1a, W1b, b1b, W2a, b2a, W2b, b2b, eps,
           Wadv, badv, Wv1, bv1, Wv2, bv2, Wv3, bv3):
    h = x.reshape(N, D)
    src = edge_index[0]
    dst = edge_index[1]
    zeros = jnp.zeros((N, D), jnp.float32)

    Ws = [(W1a, b1a.reshape(1, D), W1b, b1b.reshape(1, D))] + \
         [(W2a, b2a.reshape(1, D), W2b, b2b.reshape(1, D))] * 4

    for layer in range(5):
        agg = _segment_sum_sc(h, src, dst, zeros)
        Wa, ba, Wb, bb = Ws[layer]
        epsv = (1.0 + eps[layer]).reshape(1, 1)
        h = _mlp_tc(epsv, h, agg, Wa, ba, Wb, bb)

    h_flat = h.reshape(1, F)
    q = _head_tc(h_flat, Wadv, Wv1,
                 badv.reshape(1, NA), bv1.reshape(1, NVH),
                 Wv2, bv2.reshape(1, NVH),
                 Wv3.reshape(1, NVH), bv3.reshape(1, 1))
    return q.reshape(1, 3, 4)


# speedup vs baseline: 2.1393x; 2.1393x over previous
import functools

import jax
import jax.numpy as jnp
from jax import lax
from jax.experimental import pallas as pl
from jax.experimental.pallas import tpu as pltpu
from jax.experimental.pallas import tpu_sc as plsc

N = 10000
D = 128
E = 320000
NC = 2
NS = 16
NW = NC * NS
EPW = E // NW
CH = 120
NRING = 82
REM = EPW - (NRING + 1) * CH
RPT = 632
RPT_LAST = N - RPT * (NS - 1)


def _seg_sum_body(h_hbm, src_hbm, dst_hbm, zeros_hbm, out_hbm,
                  sidx_all, didx_all, rows0, rows1, agg_sh, sem0, sem1):
    cid = lax.axis_index("c")
    sid = lax.axis_index("s")
    wid = sid * NC + cid
    rows = (rows0, rows1)
    sems = (sem0, sem1)

    pltpu.sync_copy(src_hbm.at[pl.ds(wid * EPW, EPW)], sidx_all)
    pltpu.sync_copy(dst_hbm.at[pl.ds(wid * EPW, EPW)], didx_all)

    def gather_start(c, b):
        pltpu.async_copy(h_hbm.at[sidx_all.at[pl.ds(c * CH, CH)]],
                         rows[b], sems[b])

    def gather_wait(c, b):
        pltpu.make_async_copy(h_hbm.at[sidx_all.at[pl.ds(c * CH, CH)]],
                              rows[b], sems[b]).wait()

    gather_start(0, 0)
    gather_start(1, 1)

    @pl.when(sid < NS - 1)
    def _():
        pltpu.sync_copy(zeros_hbm.at[pl.ds(sid * RPT, RPT)],
                        agg_sh.at[pl.ds(sid * RPT, RPT)])

    @pl.when(sid == NS - 1)
    def _():
        pltpu.sync_copy(zeros_hbm.at[pl.ds(RPT * (NS - 1), RPT_LAST)],
                        agg_sh.at[pl.ds(RPT * (NS - 1), RPT_LAST)])

    plsc.subcore_barrier()

    def ring(g, _):
        for b in range(2):
            c = g + b
            gather_wait(c, b)

            @pl.when(c + 2 < NRING)
            def _():
                gather_start(c + 2, b)

            pltpu.sync_copy(rows[b], agg_sh.at[didx_all.at[pl.ds(c * CH, CH)]],
                            add=True)
        return ()

    lax.fori_loop(0, NRING // 2, lambda i, c: ring(2 * i, c), (),
                  unroll=False)

    eb = NRING * CH
    rb = eb + CH
    pltpu.async_copy(h_hbm.at[sidx_all.at[pl.ds(eb, CH)]], rows0, sem0)
    pltpu.async_copy(h_hbm.at[sidx_all.at[pl.ds(rb, REM)]],
                     rows1.at[pl.ds(0, REM)], sem1)
    pltpu.make_async_copy(h_hbm.at[sidx_all.at[pl.ds(eb, CH)]], rows0,
                          sem0).wait()
    pltpu.sync_copy(rows0, agg_sh.at[didx_all.at[pl.ds(eb, CH)]], add=True)
    pltpu.make_async_copy(h_hbm.at[sidx_all.at[pl.ds(rb, REM)]],
                          rows1.at[pl.ds(0, REM)], sem1).wait()
    pltpu.sync_copy(rows1.at[pl.ds(0, REM)],
                    agg_sh.at[didx_all.at[pl.ds(rb, REM)]], add=True)

    plsc.subcore_barrier()

    @pl.when(sid < NS - 1)
    def _():
        pltpu.sync_copy(agg_sh.at[pl.ds(sid * RPT, RPT)],
                        out_hbm.at[cid, pl.ds(sid * RPT, RPT)])

    @pl.when(sid == NS - 1)
    def _():
        pltpu.sync_copy(agg_sh.at[pl.ds(RPT * (NS - 1), RPT_LAST)],
                        out_hbm.at[cid, pl.ds(RPT * (NS - 1), RPT_LAST)])


@jax.jit
def _segment_sum_sc(h, src, dst, zeros):
    mesh = plsc.VectorSubcoreMesh(core_axis_name="c", subcore_axis_name="s",
                                  num_cores=NC, num_subcores=NS)
    f = pl.kernel(
        _seg_sum_body,
        out_type=jax.ShapeDtypeStruct((NC, N, D), jnp.float32),
        mesh=mesh,
        scratch_types=[
            pltpu.VMEM((EPW,), jnp.int32),
            pltpu.VMEM((EPW,), jnp.int32),
            pltpu.VMEM((CH, D), jnp.float32),
            pltpu.VMEM((CH, D), jnp.float32),
            pltpu.VMEM_SHARED((N, D), jnp.float32),
            pltpu.SemaphoreType.DMA,
            pltpu.SemaphoreType.DMA,
        ],
    )
    return f(h, src, dst, zeros)


BN = 1000


def _mlp_body(eps_ref, h_ref, agg_ref, Wa_ref, ba_ref, Wb_ref, bb_ref, out_ref):
    ev = eps_ref[0, 0]
    z = h_ref[...] * ev + agg_ref[0] + agg_ref[1]
    z = jnp.maximum(
        jnp.dot(z, Wa_ref[...], preferred_element_type=jnp.float32)
        + ba_ref[...], 0.0)
    z = jnp.maximum(
        jnp.dot(z, Wb_ref[...], preferred_element_type=jnp.float32)
        + bb_ref[...], 0.0)
    out_ref[...] = z


@jax.jit
def _mlp_tc(epsv, h, agg, Wa, ba, Wb, bb):
    grid = (N // BN,)
    return pl.pallas_call(
        _mlp_body,
        grid=grid,
        in_specs=[
            pl.BlockSpec(memory_space=pltpu.SMEM),
            pl.BlockSpec((BN, D), lambda i: (i, 0)),
            pl.BlockSpec((NC, BN, D), lambda i: (0, i, 0)),
            pl.BlockSpec((D, D), lambda i: (0, 0)),
            pl.BlockSpec((1, D), lambda i: (0, 0)),
            pl.BlockSpec((D, D), lambda i: (0, 0)),
            pl.BlockSpec((1, D), lambda i: (0, 0)),
        ],
        out_specs=pl.BlockSpec((BN, D), lambda i: (i, 0)),
        out_shape=jax.ShapeDtypeStruct((N, D), jnp.float32),
    )(epsv, h, agg, Wa, ba, Wb, bb)


F = N * D
BF = 6400
NBF = F // BF
NA = 12
NVH = 64


def _head_body(h_ref, WadvT_ref, Wv1T_ref, badv_ref, bv1_ref, Wv2_ref, bv2_ref,
               Wv3t_ref, bv3_ref, out_ref, acc_adv, acc_val):
    i = pl.program_id(0)

    @pl.when(i == 0)
    def _():
        acc_adv[...] = jnp.zeros_like(acc_adv)
        acc_val[...] = jnp.zeros_like(acc_val)

    hb = h_ref[...]
    acc_adv[...] += lax.dot_general(
        hb, WadvT_ref[...], (((1,), (1,)), ((), ())),
        preferred_element_type=jnp.float32)
    acc_val[...] += lax.dot_general(
        hb, Wv1T_ref[...], (((1,), (1,)), ((), ())),
        preferred_element_type=jnp.float32)

    @pl.when(i == NBF - 1)
    def _():
        adv = jnp.maximum(acc_adv[...] + badv_ref[...], 0.0)
        val = jnp.maximum(acc_val[...] + bv1_ref[...], 0.0)
        val = jnp.maximum(
            jnp.dot(val, Wv2_ref[...], preferred_element_type=jnp.float32)
            + bv2_ref[...], 0.0)
        v3 = jnp.sum(val * Wv3t_ref[...], axis=1, keepdims=True) + bv3_ref[...]
        r = lax.broadcasted_iota(jnp.int32, (NA, NA), 0)
        c = lax.broadcasted_iota(jnp.int32, (NA, NA), 1)
        G = jnp.where((r // 4) == (c // 4), 0.25, 0.0).astype(jnp.float32)
        madv = jnp.dot(adv, G, preferred_element_type=jnp.float32)
        out_ref[...] = v3 + adv - madv


@jax.jit
def _head_tc(h_flat, Wadv, Wv1, badv, bv1, Wv2, bv2, Wv3t, bv3):
    return pl.pallas_call(
        _head_body,
        grid=(NBF,),
        in_specs=[
            pl.BlockSpec((1, BF), lambda i: (0, i)),
            pl.BlockSpec((NA, BF), lambda i: (0, i)),
            pl.BlockSpec((NVH, BF), lambda i: (0, i)),
            pl.BlockSpec((1, NA), lambda i: (0, 0)),
            pl.BlockSpec((1, NVH), lambda i: (0, 0)),
            pl.BlockSpec((NVH, NVH), lambda i: (0, 0)),
            pl.BlockSpec((1, NVH), lambda i: (0, 0)),
            pl.BlockSpec((1, NVH), lambda i: (0, 0)),
            pl.BlockSpec((1, 1), lambda i: (0, 0)),
        ],
        out_specs=pl.BlockSpec((1, NA), lambda i: (0, 0)),
        out_shape=jax.ShapeDtypeStruct((1, NA), jnp.float32),
        scratch_shapes=[
            pltpu.VMEM((1, NA), jnp.float32),
            pltpu.VMEM((1, NVH), jnp.float32),
        ],
    )(h_flat, Wadv, Wv1, badv, bv1, Wv2, bv2, Wv3t, bv3)


def kernel(x, edge_index, W1a, b1a, W1b, b1b, W2a, b2a, W2b, b2b, eps,
           Wadv, badv, Wv1, bv1, Wv2, bv2, Wv3, bv3):
    h = x.reshape(N, D)
    src = edge_index[0]
    dst = edge_index[1]
    zeros = jnp.zeros((N, D), jnp.float32)

    Ws = [(W1a, b1a.reshape(1, D), W1b, b1b.reshape(1, D))] + \
         [(W2a, b2a.reshape(1, D), W2b, b2b.reshape(1, D))] * 4

    for layer in range(5):
        agg = _segment_sum_sc(h, src, dst, zeros)
        Wa, ba, Wb, bb = Ws[layer]
        epsv = (1.0 + eps[layer]).reshape(1, 1)
        h = _mlp_tc(epsv, h, agg, Wa, ba, Wb, bb)

    h_flat = h.reshape(1, F)
    q = _head_tc(h_flat, Wadv.T, Wv1.T,
                 badv.reshape(1, NA), bv1.reshape(1, NVH),
                 Wv2, bv2.reshape(1, NVH),
                 Wv3.reshape(1, NVH), bv3.reshape(1, 1))
    return q.reshape(1, 3, 4)


# speedup vs baseline: 2.1423x; 1.0014x over previous
import functools

import jax
import jax.numpy as jnp
from jax import lax
from jax.experimental import pallas as pl
from jax.experimental.pallas import tpu as pltpu
from jax.experimental.pallas import tpu_sc as plsc

N = 10000
D = 128
E = 320000
NC = 2
NS = 16
NW = NC * NS
EPW = E // NW
CH = 120
NRING = 82
REM = EPW - (NRING + 1) * CH
RPT = 632
RPT_LAST = N - RPT * (NS - 1)


def _seg_sum_body(h_hbm, src_hbm, dst_hbm, zeros_hbm, out_hbm,
                  sidx_all, didx_all, rows0, rows1, agg_sh, sem0, sem1):
    cid = lax.axis_index("c")
    sid = lax.axis_index("s")
    wid = sid * NC + cid
    rows = (rows0, rows1)
    sems = (sem0, sem1)

    pltpu.sync_copy(src_hbm.at[pl.ds(wid * EPW, EPW)], sidx_all)
    pltpu.sync_copy(dst_hbm.at[pl.ds(wid * EPW, EPW)], didx_all)

    def gather_start(c, b):
        pltpu.async_copy(h_hbm.at[sidx_all.at[pl.ds(c * CH, CH)]],
                         rows[b], sems[b])

    def gather_wait(c, b):
        pltpu.make_async_copy(h_hbm.at[sidx_all.at[pl.ds(c * CH, CH)]],
                              rows[b], sems[b]).wait()

    gather_start(0, 0)
    gather_start(1, 1)

    @pl.when(sid < NS - 1)
    def _():
        pltpu.sync_copy(zeros_hbm.at[pl.ds(sid * RPT, RPT)],
                        agg_sh.at[pl.ds(sid * RPT, RPT)])

    @pl.when(sid == NS - 1)
    def _():
        pltpu.sync_copy(zeros_hbm.at[pl.ds(RPT * (NS - 1), RPT_LAST)],
                        agg_sh.at[pl.ds(RPT * (NS - 1), RPT_LAST)])

    plsc.subcore_barrier()

    def ring(g, _):
        for b in range(2):
            c = g + b
            gather_wait(c, b)

            @pl.when(c + 2 < NRING)
            def _():
                gather_start(c + 2, b)

            pltpu.sync_copy(rows[b], agg_sh.at[didx_all.at[pl.ds(c * CH, CH)]],
                            add=True)
        return ()

    lax.fori_loop(0, NRING // 2, lambda i, c: ring(2 * i, c), (),
                  unroll=False)

    eb = NRING * CH
    rb = eb + CH
    pltpu.async_copy(h_hbm.at[sidx_all.at[pl.ds(eb, CH)]], rows0, sem0)
    pltpu.async_copy(h_hbm.at[sidx_all.at[pl.ds(rb, REM)]],
                     rows1.at[pl.ds(0, REM)], sem1)
    pltpu.make_async_copy(h_hbm.at[sidx_all.at[pl.ds(eb, CH)]], rows0,
                          sem0).wait()
    pltpu.sync_copy(rows0, agg_sh.at[didx_all.at[pl.ds(eb, CH)]], add=True)
    pltpu.make_async_copy(h_hbm.at[sidx_all.at[pl.ds(rb, REM)]],
                          rows1.at[pl.ds(0, REM)], sem1).wait()
    pltpu.sync_copy(rows1.at[pl.ds(0, REM)],
                    agg_sh.at[didx_all.at[pl.ds(rb, REM)]], add=True)

    plsc.subcore_barrier()

    @pl.when(sid < NS - 1)
    def _():
        pltpu.sync_copy(agg_sh.at[pl.ds(sid * RPT, RPT)],
                        out_hbm.at[cid, pl.ds(sid * RPT, RPT)])

    @pl.when(sid == NS - 1)
    def _():
        pltpu.sync_copy(agg_sh.at[pl.ds(RPT * (NS - 1), RPT_LAST)],
                        out_hbm.at[cid, pl.ds(RPT * (NS - 1), RPT_LAST)])


@jax.jit
def _segment_sum_sc(h, src, dst, zeros):
    mesh = plsc.VectorSubcoreMesh(core_axis_name="c", subcore_axis_name="s",
                                  num_cores=NC, num_subcores=NS)
    f = pl.kernel(
        _seg_sum_body,
        out_type=jax.ShapeDtypeStruct((NC, N, D), jnp.float32),
        mesh=mesh,
        scratch_types=[
            pltpu.VMEM((EPW,), jnp.int32),
            pltpu.VMEM((EPW,), jnp.int32),
            pltpu.VMEM((CH, D), jnp.float32),
            pltpu.VMEM((CH, D), jnp.float32),
            pltpu.VMEM_SHARED((N, D), jnp.float32),
            pltpu.SemaphoreType.DMA,
            pltpu.SemaphoreType.DMA,
        ],
    )
    return f(h, src, dst, zeros)


BN = 1000


def _mlp_body(eps_ref, h_ref, agg_ref, Wa_ref, ba_ref, Wb_ref, bb_ref, out_ref):
    ev = eps_ref[0, 0]
    z = h_ref[...] * ev + agg_ref[0] + agg_ref[1]
    z = jnp.maximum(
        jnp.dot(z, Wa_ref[...], preferred_element_type=jnp.float32)
        + ba_ref[...], 0.0)
    z = jnp.maximum(
        jnp.dot(z, Wb_ref[...], preferred_element_type=jnp.float32)
        + bb_ref[...], 0.0)
    out_ref[...] = z


@jax.jit
def _mlp_tc(epsv, h, agg, Wa, ba, Wb, bb):
    grid = (N // BN,)
    return pl.pallas_call(
        _mlp_body,
        grid=grid,
        in_specs=[
            pl.BlockSpec(memory_space=pltpu.SMEM),
            pl.BlockSpec((BN, D), lambda i: (i, 0)),
            pl.BlockSpec((NC, BN, D), lambda i: (0, i, 0)),
            pl.BlockSpec((D, D), lambda i: (0, 0)),
            pl.BlockSpec((1, D), lambda i: (0, 0)),
            pl.BlockSpec((D, D), lambda i: (0, 0)),
            pl.BlockSpec((1, D), lambda i: (0, 0)),
        ],
        out_specs=pl.BlockSpec((BN, D), lambda i: (i, 0)),
        out_shape=jax.ShapeDtypeStruct((N, D), jnp.float32),
    )(epsv, h, agg, Wa, ba, Wb, bb)


F = N * D
BF = 6400
NBF = F // BF
NA = 12
NVH = 64


def _head_body(h_ref, WadvT_ref, Wv1T_ref, badv_ref, bv1_ref, Wv2_ref, bv2_ref,
               Wv3t_ref, bv3_ref, out_ref, acc_adv, acc_val):
    i = pl.program_id(0)

    @pl.when(i == 0)
    def _():
        acc_adv[...] = jnp.zeros_like(acc_adv)
        acc_val[...] = jnp.zeros_like(acc_val)

    hb = h_ref[...]
    acc_adv[...] += lax.dot_general(
        hb, WadvT_ref[...], (((1,), (1,)), ((), ())),
        preferred_element_type=jnp.float32)
    acc_val[...] += lax.dot_general(
        hb, Wv1T_ref[...], (((1,), (1,)), ((), ())),
        preferred_element_type=jnp.float32)

    @pl.when(i == NBF - 1)
    def _():
        adv = jnp.maximum(acc_adv[...] + badv_ref[...], 0.0)
        val = jnp.maximum(acc_val[...] + bv1_ref[...], 0.0)
        val = jnp.maximum(
            jnp.dot(val, Wv2_ref[...], preferred_element_type=jnp.float32)
            + bv2_ref[...], 0.0)
        v3 = jnp.sum(val * Wv3t_ref[...], axis=1, keepdims=True) + bv3_ref[...]
        r = lax.broadcasted_iota(jnp.int32, (NA, NA), 0)
        c = lax.broadcasted_iota(jnp.int32, (NA, NA), 1)
        G = jnp.where((r // 4) == (c // 4), 0.25, 0.0).astype(jnp.float32)
        madv = jnp.dot(adv, G, preferred_element_type=jnp.float32)
        out_ref[...] = v3 + adv - madv


@jax.jit
def _head_tc(h_flat, Wadv, Wv1, badv, bv1, Wv2, bv2, Wv3t, bv3):
    return pl.pallas_call(
        _head_body,
        grid=(NBF,),
        in_specs=[
            pl.BlockSpec((1, BF), lambda i: (0, i)),
            pl.BlockSpec((NA, BF), lambda i: (0, i)),
            pl.BlockSpec((NVH, BF), lambda i: (0, i)),
            pl.BlockSpec((1, NA), lambda i: (0, 0)),
            pl.BlockSpec((1, NVH), lambda i: (0, 0)),
            pl.BlockSpec((NVH, NVH), lambda i: (0, 0)),
            pl.BlockSpec((1, NVH), lambda i: (0, 0)),
            pl.BlockSpec((1, NVH), lambda i: (0, 0)),
            pl.BlockSpec((1, 1), lambda i: (0, 0)),
        ],
        out_specs=pl.BlockSpec((1, NA), lambda i: (0, 0)),
        out_shape=jax.ShapeDtypeStruct((1, NA), jnp.float32),
        scratch_shapes=[
            pltpu.VMEM((1, NA), jnp.float32),
            pltpu.VMEM((1, NVH), jnp.float32),
        ],
    )(h_flat, Wadv, Wv1, badv, bv1, Wv2, bv2, Wv3t, bv3)


def kernel(x, edge_index, W1a, b1a, W1b, b1b, W2a, b2a, W2b, b2b, eps,
           Wadv, badv, Wv1, bv1, Wv2, bv2, Wv3, bv3):
    h = x.reshape(N, D)
    zeros = jnp.zeros((N, D), jnp.float32)
    src = edge_index[0]
    dst = edge_index[1]

    Ws = [(W1a, b1a.reshape(1, D), W1b, b1b.reshape(1, D))] + \
         [(W2a, b2a.reshape(1, D), W2b, b2b.reshape(1, D))] * 4

    for layer in range(5):
        agg = _segment_sum_sc(h, src, dst, zeros)
        Wa, ba, Wb, bb = Ws[layer]
        epsv = (1.0 + eps[layer]).reshape(1, 1)
        h = _mlp_tc(epsv, h, agg, Wa, ba, Wb, bb)

    h_flat = h.reshape(1, F)
    q = _head_tc(h_flat, Wadv.T, Wv1.T,
                 badv.reshape(1, NA), bv1.reshape(1, NVH),
                 Wv2, bv2.reshape(1, NVH),
                 Wv3.reshape(1, NVH), bv3.reshape(1, 1))
    return q.reshape(1, 3, 4)


# speedup vs baseline: 2.3991x; 1.1198x over previous
import functools

import jax
import jax.numpy as jnp
from jax import lax
from jax.experimental import pallas as pl
from jax.experimental.pallas import tpu as pltpu
from jax.experimental.pallas import tpu_sc as plsc

N = 10000
D = 128
E = 320000
NC = 2
NS = 16
NW = NC * NS
EPW = E // NW
CH = 120
NRING = 82
REM = EPW - (NRING + 1) * CH
RPT = 632
RPT_LAST = N - RPT * (NS - 1)


def _seg_sum_body(h_hbm, src_hbm, dst_hbm, zeros_hbm, out_hbm,
                  sidx_all, didx_all, rows0, rows1, agg_sh, sem0, sem1):
    cid = lax.axis_index("c")
    sid = lax.axis_index("s")
    wid = sid * NC + cid
    rows = (rows0, rows1)
    sems = (sem0, sem1)

    pltpu.sync_copy(src_hbm.at[pl.ds(wid * EPW, EPW)], sidx_all)
    pltpu.sync_copy(dst_hbm.at[pl.ds(wid * EPW, EPW)], didx_all)

    def gather_start(c, b):
        pltpu.async_copy(h_hbm.at[sidx_all.at[pl.ds(c * CH, CH)]],
                         rows[b], sems[b])

    def gather_wait(c, b):
        pltpu.make_async_copy(h_hbm.at[sidx_all.at[pl.ds(c * CH, CH)]],
                              rows[b], sems[b]).wait()

    gather_start(0, 0)
    gather_start(1, 1)

    @pl.when(sid < NS - 1)
    def _():
        pltpu.sync_copy(zeros_hbm.at[pl.ds(sid * RPT, RPT)],
                        agg_sh.at[pl.ds(sid * RPT, RPT)])

    @pl.when(sid == NS - 1)
    def _():
        pltpu.sync_copy(zeros_hbm.at[pl.ds(RPT * (NS - 1), RPT_LAST)],
                        agg_sh.at[pl.ds(RPT * (NS - 1), RPT_LAST)])

    plsc.subcore_barrier()

    def ring(g, _):
        for b in range(2):
            c = g + b
            gather_wait(c, b)

            @pl.when(c + 2 < NRING)
            def _():
                gather_start(c + 2, b)

            pltpu.sync_copy(rows[b], agg_sh.at[didx_all.at[pl.ds(c * CH, CH)]],
                            add=True)
        return ()

    lax.fori_loop(0, NRING // 2, lambda i, c: ring(2 * i, c), (),
                  unroll=False)

    eb = NRING * CH
    rb = eb + CH
    pltpu.async_copy(h_hbm.at[sidx_all.at[pl.ds(eb, CH)]], rows0, sem0)
    pltpu.async_copy(h_hbm.at[sidx_all.at[pl.ds(rb, REM)]],
                     rows1.at[pl.ds(0, REM)], sem1)
    pltpu.make_async_copy(h_hbm.at[sidx_all.at[pl.ds(eb, CH)]], rows0,
                          sem0).wait()
    pltpu.sync_copy(rows0, agg_sh.at[didx_all.at[pl.ds(eb, CH)]], add=True)
    pltpu.make_async_copy(h_hbm.at[sidx_all.at[pl.ds(rb, REM)]],
                          rows1.at[pl.ds(0, REM)], sem1).wait()
    pltpu.sync_copy(rows1.at[pl.ds(0, REM)],
                    agg_sh.at[didx_all.at[pl.ds(rb, REM)]], add=True)

    plsc.subcore_barrier()

    @pl.when(sid < NS - 1)
    def _():
        pltpu.sync_copy(agg_sh.at[pl.ds(sid * RPT, RPT)],
                        out_hbm.at[cid, pl.ds(sid * RPT, RPT)])

    @pl.when(sid == NS - 1)
    def _():
        pltpu.sync_copy(agg_sh.at[pl.ds(RPT * (NS - 1), RPT_LAST)],
                        out_hbm.at[cid, pl.ds(RPT * (NS - 1), RPT_LAST)])


@jax.jit
def _segment_sum_sc(h, src, dst, zeros):
    mesh = plsc.VectorSubcoreMesh(core_axis_name="c", subcore_axis_name="s",
                                  num_cores=NC, num_subcores=NS)
    f = pl.kernel(
        _seg_sum_body,
        out_type=jax.ShapeDtypeStruct((NC, N, D), jnp.float32),
        mesh=mesh,
        scratch_types=[
            pltpu.VMEM((EPW,), jnp.int32),
            pltpu.VMEM((EPW,), jnp.int32),
            pltpu.VMEM((CH, D), jnp.float32),
            pltpu.VMEM((CH, D), jnp.float32),
            pltpu.VMEM_SHARED((N, D), jnp.float32),
            pltpu.SemaphoreType.DMA,
            pltpu.SemaphoreType.DMA,
        ],
    )
    return f(h, src, dst, zeros)


BN = 1000


def _mlp_body(eps_ref, h_ref, agg_ref, Wa_ref, ba_ref, Wb_ref, bb_ref, out_ref):
    ev = eps_ref[0, 0]
    z = h_ref[...] * ev + agg_ref[0] + agg_ref[1]
    z = jnp.maximum(
        jnp.dot(z, Wa_ref[...], preferred_element_type=jnp.float32)
        + ba_ref[...], 0.0)
    z = jnp.maximum(
        jnp.dot(z, Wb_ref[...], preferred_element_type=jnp.float32)
        + bb_ref[...], 0.0)
    out_ref[...] = z


@jax.jit
def _mlp_tc(epsv, h, agg, Wa, ba, Wb, bb):
    grid = (N // BN,)
    return pl.pallas_call(
        _mlp_body,
        grid=grid,
        in_specs=[
            pl.BlockSpec(memory_space=pltpu.SMEM),
            pl.BlockSpec((BN, D), lambda i: (i, 0)),
            pl.BlockSpec((NC, BN, D), lambda i: (0, i, 0)),
            pl.BlockSpec((D, D), lambda i: (0, 0)),
            pl.BlockSpec((1, D), lambda i: (0, 0)),
            pl.BlockSpec((D, D), lambda i: (0, 0)),
            pl.BlockSpec((1, D), lambda i: (0, 0)),
        ],
        out_specs=pl.BlockSpec((BN, D), lambda i: (i, 0)),
        out_shape=jax.ShapeDtypeStruct((N, D), jnp.float32),
    )(epsv, h, agg, Wa, ba, Wb, bb)


F = N * D
BF = 25600
NBF = F // BF
NA = 12
NVH = 64


def _head_body(h_ref, WadvT_ref, Wv1T_ref, badv_ref, bv1_ref, Wv2_ref, bv2_ref,
               Wv3t_ref, bv3_ref, out_ref, acc_adv, acc_val):
    i = pl.program_id(0)

    @pl.when(i == 0)
    def _():
        acc_adv[...] = jnp.zeros_like(acc_adv)
        acc_val[...] = jnp.zeros_like(acc_val)

    hb = h_ref[...]
    acc_adv[...] += lax.dot_general(
        hb, WadvT_ref[...], (((1,), (1,)), ((), ())),
        preferred_element_type=jnp.float32)
    acc_val[...] += lax.dot_general(
        hb, Wv1T_ref[...], (((1,), (1,)), ((), ())),
        preferred_element_type=jnp.float32)

    @pl.when(i == NBF - 1)
    def _():
        adv = jnp.maximum(acc_adv[...] + badv_ref[...], 0.0)
        val = jnp.maximum(acc_val[...] + bv1_ref[...], 0.0)
        val = jnp.maximum(
            jnp.dot(val, Wv2_ref[...], preferred_element_type=jnp.float32)
            + bv2_ref[...], 0.0)
        v3 = jnp.sum(val * Wv3t_ref[...], axis=1, keepdims=True) + bv3_ref[...]
        r = lax.broadcasted_iota(jnp.int32, (NA, NA), 0)
        c = lax.broadcasted_iota(jnp.int32, (NA, NA), 1)
        G = jnp.where((r // 4) == (c // 4), 0.25, 0.0).astype(jnp.float32)
        madv = jnp.dot(adv, G, preferred_element_type=jnp.float32)
        out_ref[...] = v3 + adv - madv


@jax.jit
def _head_tc(h_flat, Wadv, Wv1, badv, bv1, Wv2, bv2, Wv3t, bv3):
    return pl.pallas_call(
        _head_body,
        grid=(NBF,),
        in_specs=[
            pl.BlockSpec((1, BF), lambda i: (0, i)),
            pl.BlockSpec((NA, BF), lambda i: (0, i)),
            pl.BlockSpec((NVH, BF), lambda i: (0, i)),
            pl.BlockSpec((1, NA), lambda i: (0, 0)),
            pl.BlockSpec((1, NVH), lambda i: (0, 0)),
            pl.BlockSpec((NVH, NVH), lambda i: (0, 0)),
            pl.BlockSpec((1, NVH), lambda i: (0, 0)),
            pl.BlockSpec((1, NVH), lambda i: (0, 0)),
            pl.BlockSpec((1, 1), lambda i: (0, 0)),
        ],
        out_specs=pl.BlockSpec((1, NA), lambda i: (0, 0)),
        out_shape=jax.ShapeDtypeStruct((1, NA), jnp.float32),
        scratch_shapes=[
            pltpu.VMEM((1, NA), jnp.float32),
            pltpu.VMEM((1, NVH), jnp.float32),
        ],
    )(h_flat, Wadv, Wv1, badv, bv1, Wv2, bv2, Wv3t, bv3)


def kernel(x, edge_index, W1a, b1a, W1b, b1b, W2a, b2a, W2b, b2b, eps,
           Wadv, badv, Wv1, bv1, Wv2, bv2, Wv3, bv3):
    h = x.reshape(N, D)
    zeros = jnp.zeros((N, D), jnp.float32)
    src = edge_index[0]
    dst = edge_index[1]

    Ws = [(W1a, b1a.reshape(1, D), W1b, b1b.reshape(1, D))] + \
         [(W2a, b2a.reshape(1, D), W2b, b2b.reshape(1, D))] * 4

    for layer in range(5):
        agg = _segment_sum_sc(h, src, dst, zeros)
        Wa, ba, Wb, bb = Ws[layer]
        epsv = (1.0 + eps[layer]).reshape(1, 1)
        h = _mlp_tc(epsv, h, agg, Wa, ba, Wb, bb)

    h_flat = h.reshape(1, F)
    q = _head_tc(h_flat, Wadv.T, Wv1.T,
                 badv.reshape(1, NA), bv1.reshape(1, NVH),
                 Wv2, bv2.reshape(1, NVH),
                 Wv3.reshape(1, NVH), bv3.reshape(1, 1))
    return q.reshape(1, 3, 4)


# speedup vs baseline: 2.4070x; 1.0033x over previous
import functools

import jax
import jax.numpy as jnp
from jax import lax
from jax.experimental import pallas as pl
from jax.experimental.pallas import tpu as pltpu
from jax.experimental.pallas import tpu_sc as plsc

N = 10000
D = 128
E = 320000
NC = 2
NS = 16
NW = NC * NS
EPW = E // NW
CH = 120
NRING = 82
REM = EPW - (NRING + 1) * CH
RPT = 632
RPT_LAST = N - RPT * (NS - 1)


def _seg_sum_body(h_hbm, src_hbm, dst_hbm, zeros_hbm, out_hbm,
                  sidx_all, didx_all, rows0, rows1, agg_sh, sem0, sem1):
    cid = lax.axis_index("c")
    sid = lax.axis_index("s")
    wid = sid * NC + cid
    rows = (rows0, rows1)
    sems = (sem0, sem1)

    pltpu.sync_copy(src_hbm.at[pl.ds(wid * EPW, EPW)], sidx_all)
    pltpu.sync_copy(dst_hbm.at[pl.ds(wid * EPW, EPW)], didx_all)

    def gather_start(c, b):
        pltpu.async_copy(h_hbm.at[sidx_all.at[pl.ds(c * CH, CH)]],
                         rows[b], sems[b])

    def gather_wait(c, b):
        pltpu.make_async_copy(h_hbm.at[sidx_all.at[pl.ds(c * CH, CH)]],
                              rows[b], sems[b]).wait()

    gather_start(0, 0)
    gather_start(1, 1)

    @pl.when(sid < NS - 1)
    def _():
        pltpu.sync_copy(zeros_hbm.at[pl.ds(sid * RPT, RPT)],
                        agg_sh.at[pl.ds(sid * RPT, RPT)])

    @pl.when(sid == NS - 1)
    def _():
        pltpu.sync_copy(zeros_hbm.at[pl.ds(RPT * (NS - 1), RPT_LAST)],
                        agg_sh.at[pl.ds(RPT * (NS - 1), RPT_LAST)])

    plsc.subcore_barrier()

    def ring(g, _):
        for b in range(2):
            c = g + b
            gather_wait(c, b)

            @pl.when(c + 2 < NRING)
            def _():
                gather_start(c + 2, b)

            pltpu.sync_copy(rows[b], agg_sh.at[didx_all.at[pl.ds(c * CH, CH)]],
                            add=True)
        return ()

    lax.fori_loop(0, NRING // 2, lambda i, c: ring(2 * i, c), (),
                  unroll=False)

    eb = NRING * CH
    rb = eb + CH
    pltpu.async_copy(h_hbm.at[sidx_all.at[pl.ds(eb, CH)]], rows0, sem0)
    pltpu.async_copy(h_hbm.at[sidx_all.at[pl.ds(rb, REM)]],
                     rows1.at[pl.ds(0, REM)], sem1)
    pltpu.make_async_copy(h_hbm.at[sidx_all.at[pl.ds(eb, CH)]], rows0,
                          sem0).wait()
    pltpu.sync_copy(rows0, agg_sh.at[didx_all.at[pl.ds(eb, CH)]], add=True)
    pltpu.make_async_copy(h_hbm.at[sidx_all.at[pl.ds(rb, REM)]],
                          rows1.at[pl.ds(0, REM)], sem1).wait()
    pltpu.sync_copy(rows1.at[pl.ds(0, REM)],
                    agg_sh.at[didx_all.at[pl.ds(rb, REM)]], add=True)

    plsc.subcore_barrier()

    @pl.when(sid < NS - 1)
    def _():
        pltpu.sync_copy(agg_sh.at[pl.ds(sid * RPT, RPT)],
                        out_hbm.at[cid, pl.ds(sid * RPT, RPT)])

    @pl.when(sid == NS - 1)
    def _():
        pltpu.sync_copy(agg_sh.at[pl.ds(RPT * (NS - 1), RPT_LAST)],
                        out_hbm.at[cid, pl.ds(RPT * (NS - 1), RPT_LAST)])


@jax.jit
def _segment_sum_sc(h, src, dst, zeros):
    mesh = plsc.VectorSubcoreMesh(core_axis_name="c", subcore_axis_name="s",
                                  num_cores=NC, num_subcores=NS)
    f = pl.kernel(
        _seg_sum_body,
        out_type=jax.ShapeDtypeStruct((NC, N, D), jnp.float32),
        mesh=mesh,
        scratch_types=[
            pltpu.VMEM((EPW,), jnp.int32),
            pltpu.VMEM((EPW,), jnp.int32),
            pltpu.VMEM((CH, D), jnp.float32),
            pltpu.VMEM((CH, D), jnp.float32),
            pltpu.VMEM_SHARED((N, D), jnp.float32),
            pltpu.SemaphoreType.DMA,
            pltpu.SemaphoreType.DMA,
        ],
    )
    return f(h, src, dst, zeros)


BN = 1000


def _mlp_body(eps_ref, h_ref, agg_ref, Wa_ref, ba_ref, Wb_ref, bb_ref, out_ref):
    ev = eps_ref[0, 0]
    z = h_ref[...] * ev + agg_ref[0] + agg_ref[1]
    z = jnp.maximum(
        jnp.dot(z, Wa_ref[...], preferred_element_type=jnp.float32)
        + ba_ref[...], 0.0)
    z = jnp.maximum(
        jnp.dot(z, Wb_ref[...], preferred_element_type=jnp.float32)
        + bb_ref[...], 0.0)
    out_ref[...] = z


@jax.jit
def _mlp_tc(epsv, h, agg, Wa, ba, Wb, bb):
    grid = (N // BN,)
    return pl.pallas_call(
        _mlp_body,
        grid=grid,
        in_specs=[
            pl.BlockSpec(memory_space=pltpu.SMEM),
            pl.BlockSpec((BN, D), lambda i: (i, 0)),
            pl.BlockSpec((NC, BN, D), lambda i: (0, i, 0)),
            pl.BlockSpec((D, D), lambda i: (0, 0)),
            pl.BlockSpec((1, D), lambda i: (0, 0)),
            pl.BlockSpec((D, D), lambda i: (0, 0)),
            pl.BlockSpec((1, D), lambda i: (0, 0)),
        ],
        out_specs=pl.BlockSpec((BN, D), lambda i: (i, 0)),
        out_shape=jax.ShapeDtypeStruct((N, D), jnp.float32),
    )(epsv, h, agg, Wa, ba, Wb, bb)


F = N * D
BF = 64000
NBF = F // BF
NA = 12
NVH = 64


def _head_body(h_ref, WadvT_ref, Wv1T_ref, badv_ref, bv1_ref, Wv2_ref, bv2_ref,
               Wv3t_ref, bv3_ref, out_ref, acc_adv, acc_val):
    i = pl.program_id(0)

    @pl.when(i == 0)
    def _():
        acc_adv[...] = jnp.zeros_like(acc_adv)
        acc_val[...] = jnp.zeros_like(acc_val)

    hb = h_ref[...]
    acc_adv[...] += lax.dot_general(
        hb, WadvT_ref[...], (((1,), (1,)), ((), ())),
        preferred_element_type=jnp.float32)
    acc_val[...] += lax.dot_general(
        hb, Wv1T_ref[...], (((1,), (1,)), ((), ())),
        preferred_element_type=jnp.float32)

    @pl.when(i == NBF - 1)
    def _():
        adv = jnp.maximum(acc_adv[...] + badv_ref[...], 0.0)
        val = jnp.maximum(acc_val[...] + bv1_ref[...], 0.0)
        val = jnp.maximum(
            jnp.dot(val, Wv2_ref[...], preferred_element_type=jnp.float32)
            + bv2_ref[...], 0.0)
        v3 = jnp.sum(val * Wv3t_ref[...], axis=1, keepdims=True) + bv3_ref[...]
        r = lax.broadcasted_iota(jnp.int32, (NA, NA), 0)
        c = lax.broadcasted_iota(jnp.int32, (NA, NA), 1)
        G = jnp.where((r // 4) == (c // 4), 0.25, 0.0).astype(jnp.float32)
        madv = jnp.dot(adv, G, preferred_element_type=jnp.float32)
        out_ref[...] = v3 + adv - madv


@jax.jit
def _head_tc(h_flat, Wadv, Wv1, badv, bv1, Wv2, bv2, Wv3t, bv3):
    return pl.pallas_call(
        _head_body,
        grid=(NBF,),
        in_specs=[
            pl.BlockSpec((1, BF), lambda i: (0, i)),
            pl.BlockSpec((NA, BF), lambda i: (0, i)),
            pl.BlockSpec((NVH, BF), lambda i: (0, i)),
            pl.BlockSpec((1, NA), lambda i: (0, 0)),
            pl.BlockSpec((1, NVH), lambda i: (0, 0)),
            pl.BlockSpec((NVH, NVH), lambda i: (0, 0)),
            pl.BlockSpec((1, NVH), lambda i: (0, 0)),
            pl.BlockSpec((1, NVH), lambda i: (0, 0)),
            pl.BlockSpec((1, 1), lambda i: (0, 0)),
        ],
        out_specs=pl.BlockSpec((1, NA), lambda i: (0, 0)),
        out_shape=jax.ShapeDtypeStruct((1, NA), jnp.float32),
        scratch_shapes=[
            pltpu.VMEM((1, NA), jnp.float32),
            pltpu.VMEM((1, NVH), jnp.float32),
        ],
    )(h_flat, Wadv, Wv1, badv, bv1, Wv2, bv2, Wv3t, bv3)


def kernel(x, edge_index, W1a, b1a, W1b, b1b, W2a, b2a, W2b, b2b, eps,
           Wadv, badv, Wv1, bv1, Wv2, bv2, Wv3, bv3):
    h = x.reshape(N, D)
    zeros = jnp.zeros((N, D), jnp.float32)
    src = edge_index[0]
    dst = edge_index[1]

    Ws = [(W1a, b1a.reshape(1, D), W1b, b1b.reshape(1, D))] + \
         [(W2a, b2a.reshape(1, D), W2b, b2b.reshape(1, D))] * 4

    for layer in range(5):
        agg = _segment_sum_sc(h, src, dst, zeros)
        Wa, ba, Wb, bb = Ws[layer]
        epsv = (1.0 + eps[layer]).reshape(1, 1)
        h = _mlp_tc(epsv, h, agg, Wa, ba, Wb, bb)

    h_flat = h.reshape(1, F)
    q = _head_tc(h_flat, Wadv.T, Wv1.T,
                 badv.reshape(1, NA), bv1.reshape(1, NVH),
                 Wv2, bv2.reshape(1, NVH),
                 Wv3.reshape(1, NVH), bv3.reshape(1, 1))
    return q.reshape(1, 3, 4)
